# Initial kernel scaffold; baseline (speedup 1.0000x reference)
import jax
import jax.numpy as jnp
from jax.experimental import pallas as pl


def kernel(xyz):
    raise NotImplementedError("write your pallas kernel here")


# speedup vs baseline: 6.0393x; 6.0393x over previous
import jax
import jax.numpy as jnp
from jax import lax
from jax.experimental import pallas as pl
from jax.experimental.pallas import tpu as pltpu

_NUM_GROUP = 512
_GROUP_SIZE = 32
_ROW_BLOCK = 128


def _fps_body(xyz3_ref, cent_ref, dist_ref):
    x = xyz3_ref[0]
    y = xyz3_ref[1]
    z = xyz3_ref[2]
    b, n = x.shape
    iota = lax.broadcasted_iota(jnp.int32, (b, n), 1)
    dist_ref[...] = jnp.full((b, n), 1e10, jnp.float32)

    def step(i, far):
        onehot = iota == far
        cx = jnp.sum(jnp.where(onehot, x, 0.0), axis=1, keepdims=True)
        cy = jnp.sum(jnp.where(onehot, y, 0.0), axis=1, keepdims=True)
        cz = jnp.sum(jnp.where(onehot, z, 0.0), axis=1, keepdims=True)
        cent_ref[pl.ds(i, 1)] = jnp.concatenate([cx, cy, cz], axis=1)[None]
        dx = x - cx
        dy = y - cy
        dz = z - cz
        d = dx * dx + dy * dy + dz * dz
        dist = jnp.minimum(dist_ref[...], d)
        dist_ref[...] = dist
        m = jnp.max(dist, axis=1, keepdims=True)
        far_new = jnp.min(jnp.where(dist == m, iota, n), axis=1, keepdims=True)
        return far_new

    lax.fori_loop(0, _NUM_GROUP, step, jnp.zeros((b, 1), jnp.int32))


def _knn_body(xyzt_ref, cent_ref, nx_ref, ny_ref, nz_ref, d2_ref):
    xr = xyzt_ref[0, 0:1, :]
    yr = xyzt_ref[0, 1:2, :]
    zr = xyzt_ref[0, 2:3, :]
    c = cent_ref[0]
    cx = c[:, 0:1]
    cy = c[:, 1:2]
    cz = c[:, 2:3]
    q2 = cx * cx + cy * cy + cz * cz
    k2 = xr * xr + yr * yr + zr * zr
    bf = jnp.bfloat16
    f32 = jnp.float32
    xb = xr.astype(bf).astype(f32)
    yb = yr.astype(bf).astype(f32)
    zb = zr.astype(bf).astype(f32)
    cxb = cx.astype(bf).astype(f32)
    cyb = cy.astype(bf).astype(f32)
    czb = cz.astype(bf).astype(f32)
    qk = cxb * xb + cyb * yb + czb * zb
    d2_ref[...] = q2 + k2 - 2.0 * qk
    r, n = d2_ref.shape
    iota = lax.broadcasted_iota(jnp.int32, (r, n), 1)

    for j in range(_GROUP_SIZE):
        d2 = d2_ref[...]
        m = jnp.min(d2, axis=1, keepdims=True)
        idx = jnp.min(jnp.where(d2 == m, iota, n), axis=1, keepdims=True)
        onehot = iota == idx
        d2_ref[...] = jnp.where(onehot, jnp.inf, d2)
        gx = jnp.sum(jnp.where(onehot, xr, 0.0), axis=1, keepdims=True)
        gy = jnp.sum(jnp.where(onehot, yr, 0.0), axis=1, keepdims=True)
        gz = jnp.sum(jnp.where(onehot, zr, 0.0), axis=1, keepdims=True)
        nx_ref[0, :, j:j + 1] = gx - cx
        ny_ref[0, :, j:j + 1] = gy - cy
        nz_ref[0, :, j:j + 1] = gz - cz


def kernel(xyz):
    b, n, _ = xyz.shape
    g, k, r = _NUM_GROUP, _GROUP_SIZE, _ROW_BLOCK
    xyz3 = jnp.transpose(xyz, (2, 0, 1))

    cent_gb3 = pl.pallas_call(
        _fps_body,
        out_shape=jax.ShapeDtypeStruct((g, b, 3), jnp.float32),
        scratch_shapes=[pltpu.VMEM((b, n), jnp.float32)],
        interpret=False,
    )(xyz3)

    center = jnp.transpose(cent_gb3, (1, 0, 2))
    xyzt = jnp.transpose(xyz, (0, 2, 1))

    nbr_shape = jax.ShapeDtypeStruct((b, g, k), jnp.float32)
    nx, ny, nz = pl.pallas_call(
        _knn_body,
        grid=(b, g // r),
        in_specs=[
            pl.BlockSpec((1, 3, n), lambda i, j: (i, 0, 0)),
            pl.BlockSpec((1, r, 3), lambda i, j: (i, j, 0)),
        ],
        out_specs=[
            pl.BlockSpec((1, r, k), lambda i, j: (i, j, 0)),
            pl.BlockSpec((1, r, k), lambda i, j: (i, j, 0)),
            pl.BlockSpec((1, r, k), lambda i, j: (i, j, 0)),
        ],
        out_shape=(nbr_shape, nbr_shape, nbr_shape),
        scratch_shapes=[pltpu.VMEM((r, n), jnp.float32)],
        interpret=False,
    )(xyzt, center)

    neighborhood = jnp.stack([nx, ny, nz], axis=-1)
    return neighborhood, center


# speedup vs baseline: 9.1020x; 1.5071x over previous
import functools

import jax
import jax.numpy as jnp
from jax import lax
from jax.experimental import pallas as pl
from jax.experimental.pallas import tpu as pltpu
from jax.experimental.pallas import tpu_sc as plsc

_NUM_GROUP = 512
_GROUP_SIZE = 32
_ROW_BLOCK = 128
_NUM_SC_WORKERS = 32


def _fps_body(xyz3_ref, cent_ref, dist_ref):
    x = xyz3_ref[0]
    y = xyz3_ref[1]
    z = xyz3_ref[2]
    b, n = x.shape
    iota = lax.broadcasted_iota(jnp.int32, (b, n), 1)
    dist_ref[...] = jnp.full((b, n), 1e10, jnp.float32)

    def step(i, far):
        onehot = iota == far
        cx = jnp.sum(jnp.where(onehot, x, 0.0), axis=1, keepdims=True)
        cy = jnp.sum(jnp.where(onehot, y, 0.0), axis=1, keepdims=True)
        cz = jnp.sum(jnp.where(onehot, z, 0.0), axis=1, keepdims=True)
        cent_ref[pl.ds(i, 1)] = jnp.concatenate([cx, cy, cz], axis=1)[None]
        dx = x - cx
        dy = y - cy
        dz = z - cz
        d = dx * dx + dy * dy + dz * dz
        dist = jnp.minimum(dist_ref[...], d)
        dist_ref[...] = dist
        m = jnp.max(dist, axis=1, keepdims=True)
        far_new = jnp.min(jnp.where(dist == m, iota, n), axis=1, keepdims=True)
        return far_new

    lax.fori_loop(0, _NUM_GROUP, step, jnp.zeros((b, 1), jnp.int32))


def _knn_body(g_total, xyzt_ref, cent_ref, gi_ref, rep_ref, d2_ref):
    xr = xyzt_ref[0, 0:1, :]
    yr = xyzt_ref[0, 1:2, :]
    zr = xyzt_ref[0, 2:3, :]
    c = cent_ref[0]
    cx = c[:, 0:1]
    cy = c[:, 1:2]
    cz = c[:, 2:3]
    q2 = cx * cx + cy * cy + cz * cz
    k2 = xr * xr + yr * yr + zr * zr
    bf = jnp.bfloat16
    f32 = jnp.float32
    xb = xr.astype(bf).astype(f32)
    yb = yr.astype(bf).astype(f32)
    zb = zr.astype(bf).astype(f32)
    cxb = cx.astype(bf).astype(f32)
    cyb = cy.astype(bf).astype(f32)
    czb = cz.astype(bf).astype(f32)
    qk = cxb * xb + cyb * yb + czb * zb
    d2_ref[...] = q2 + k2 - 2.0 * qk
    r, n = d2_ref.shape
    iota = lax.broadcasted_iota(jnp.int32, (r, n), 1)
    bi = pl.program_id(0)
    ji = pl.program_id(1)
    row0 = bi * g_total + ji * r
    rep_ref[0] = row0 + lax.broadcasted_iota(jnp.int32, (r, _GROUP_SIZE), 0)
    base = bi * n

    for j in range(_GROUP_SIZE):
        d2 = d2_ref[...]
        m = jnp.min(d2, axis=1, keepdims=True)
        idx = jnp.min(jnp.where(d2 == m, iota, n), axis=1, keepdims=True)
        d2_ref[...] = jnp.where(iota == idx, jnp.inf, d2)
        gi_ref[0, :, j:j + 1] = idx + base


def _group_sc_body(gidx_ref, grep_ref, xyz_ref, cent_ref, pts_ref, cex_ref,
                   idx_v, rep_v, buf_v, sem1):
    b_total, wpb, nchunks, cw = gidx_ref.shape
    wid = lax.axis_index("s") * 2 + lax.axis_index("c")
    b = wid // wpb
    h = wid % wpb
    pltpu.sync_copy(gidx_ref.at[b, h], idx_v)
    pltpu.sync_copy(grep_ref.at[b, h], rep_v)
    span = nchunks * cw
    bufrows = buf_v.shape[0]
    cpb = bufrows // cw

    def fill(table_ref, ix_v, out_hbm_ref):
        def half(hh, carry):
            def chunk(ci, carry2):
                dst = buf_v.at[pl.ds(ci * cw, cw)]
                pltpu.async_copy(table_ref.at[ix_v.at[hh * cpb + ci]],
                                 dst, sem1).wait()
                return carry2

            lax.fori_loop(0, cpb, chunk, 0)
            pltpu.sync_copy(
                buf_v, out_hbm_ref.at[b, pl.ds(h * span + hh * bufrows,
                                               bufrows)])
            return carry

        lax.fori_loop(0, nchunks // cpb, half, 0)

    fill(xyz_ref, idx_v, pts_ref)
    fill(cent_ref, rep_v, cex_ref)


def _sub_body(a_ref, b_ref, o_ref):
    o_ref[...] = a_ref[...] - b_ref[...]


def kernel(xyz):
    b, n, _ = xyz.shape
    g, k, r = _NUM_GROUP, _GROUP_SIZE, _ROW_BLOCK
    xyz3 = jnp.transpose(xyz, (2, 0, 1))

    cent_gb3 = pl.pallas_call(
        _fps_body,
        out_shape=jax.ShapeDtypeStruct((g, b, 3), jnp.float32),
        scratch_shapes=[pltpu.VMEM((b, n), jnp.float32)],
        interpret=False,
    )(xyz3)

    center = jnp.transpose(cent_gb3, (1, 0, 2))
    xyzt = jnp.transpose(xyz, (0, 2, 1))

    idx_shape = jax.ShapeDtypeStruct((b, g, k), jnp.int32)
    gidx, grep = pl.pallas_call(
        functools.partial(_knn_body, g),
        grid=(b, g // r),
        in_specs=[
            pl.BlockSpec((1, 3, n), lambda i, j: (i, 0, 0)),
            pl.BlockSpec((1, r, 3), lambda i, j: (i, j, 0)),
        ],
        out_specs=[
            pl.BlockSpec((1, r, k), lambda i, j: (i, j, 0)),
            pl.BlockSpec((1, r, k), lambda i, j: (i, j, 0)),
        ],
        out_shape=(idx_shape, idx_shape),
        scratch_shapes=[pltpu.VMEM((r, n), jnp.float32)],
        interpret=False,
    )(xyzt, center)

    wpb = _NUM_SC_WORKERS // b
    cw = 128
    wide = 16
    nchunks = g * k // (wpb * cw)
    span = nchunks * cw
    pad = ((0, 0), (0, wide - 3))
    xyz_wide = jnp.pad(xyz.reshape(b * n, 3), pad)
    cent_wide = jnp.pad(center.reshape(b * g, 3), pad)
    gather_shape = jax.ShapeDtypeStruct((b, wpb * span, wide), jnp.float32)
    pts, cex = pl.kernel(
        _group_sc_body,
        out_type=(gather_shape, gather_shape),
        mesh=plsc.VectorSubcoreMesh(core_axis_name="c", subcore_axis_name="s"),
        compiler_params=pltpu.CompilerParams(use_tc_tiling_on_sc=False),
        scratch_types=[
            pltpu.VMEM((nchunks, cw), jnp.int32),
            pltpu.VMEM((nchunks, cw), jnp.int32),
            pltpu.VMEM((span // 2, wide), jnp.float32),
            pltpu.SemaphoreType.DMA,
        ],
        interpret=False,
    )(gidx.reshape(b, wpb, nchunks, cw), grep.reshape(b, wpb, nchunks, cw),
      xyz_wide, cent_wide)

    rows = b * g * k * wide // 128
    rb = 2048
    nbh_wide = pl.pallas_call(
        _sub_body,
        grid=(rows // rb,),
        in_specs=[
            pl.BlockSpec((rb, 128), lambda i: (i, 0)),
            pl.BlockSpec((rb, 128), lambda i: (i, 0)),
        ],
        out_specs=pl.BlockSpec((rb, 128), lambda i: (i, 0)),
        out_shape=jax.ShapeDtypeStruct((rows, 128), jnp.float32),
        interpret=False,
    )(pts.reshape(rows, 128), cex.reshape(rows, 128))

    nbh = nbh_wide.reshape(b, g * k, wide)[:, :, :3]
    return nbh.reshape(b, g, k, 3), center
